# baseline (speedup 1.0000x reference)
import functools

import jax
import jax.numpy as jnp
from jax import lax
from jax.experimental import pallas as pl
from jax.experimental.pallas import tpu as pltpu
from jax.experimental.pallas import tpu_sc as plsc

_NC = 2
_NS = 16
_NW = _NC * _NS
_LANES = 16
_NBUF = 5
_AHEAD = 3


def _sc_add(B, L, D):
    rows_per_w = L // _NW
    phase_rows = rows_per_w // 2
    sub = 16
    per_phase = B * (phase_rows // sub)
    n_steps = 2 * per_phase
    cgroups = D // _LANES
    mesh = plsc.VectorSubcoreMesh(core_axis_name="c", subcore_axis_name="s")

    @functools.partial(
        pl.kernel,
        mesh=mesh,
        out_type=jax.ShapeDtypeStruct((B * L, D), jnp.float32),
        scratch_types=[
            pltpu.VMEM((phase_rows, D), jnp.float32),
            [pltpu.VMEM((sub, D), jnp.float32) for _ in range(_NBUF)],
            pltpu.SemaphoreType.DMA,
            [pltpu.SemaphoreType.DMA for _ in range(_NBUF)],
            [pltpu.SemaphoreType.DMA for _ in range(_NBUF)],
        ],
    )
    def k(x_hbm, t_hbm, o_hbm, tch, bufs, sem_t, sems_in, sems_out):
        wid = lax.axis_index("s") * _NC + lax.axis_index("c")
        l0 = wid * rows_per_w
        n_sub_p = phase_rows // sub

        def hbm_row(i):
            p, j = divmod(i, per_phase)
            b, s = divmod(j, n_sub_p)
            return b * L + l0 + p * phase_rows + s * sub

        def issue_in(t):
            return pltpu.async_copy(
                x_hbm.at[pl.ds(hbm_row(t), sub)],
                bufs[t % _NBUF],
                sems_in[t % _NBUF],
            )

        def load_table(p):
            return pltpu.async_copy(
                t_hbm.at[pl.ds(l0 + p * phase_rows, phase_rows)], tch, sem_t
            )

        t_dma = load_table(0)
        pend_in = {}
        pend_out = {}
        for j in range(_AHEAD):
            pend_in[j] = issue_in(j)
        t_dma.wait()

        for i in range(n_steps):
            t = i + _AHEAD
            if t < n_steps:
                if t >= _NBUF:
                    pend_out[t - _NBUF].wait()
                pend_in[t] = issue_in(t)
            if i == per_phase:
                load_table(1).wait()
            pend_in[i].wait()
            xb = bufs[i % _NBUF]
            trow0 = ((i % per_phase) % n_sub_p) * sub

            def add_body(j, xb=xb, trow0=trow0):
                r = j // cgroups
                c = (j % cgroups) * _LANES
                plsc.addupdate(
                    xb.at[r, pl.ds(c, _LANES)],
                    tch[trow0 + r, pl.ds(c, _LANES)],
                )

            plsc.parallel_loop(0, sub * cgroups, unroll=8)(add_body)
            pend_out[i] = pltpu.async_copy(
                xb, o_hbm.at[pl.ds(hbm_row(i), sub)], sems_out[i % _NBUF]
            )

        for i in range(n_steps - _NBUF, n_steps):
            pend_out[i].wait()

    return k


def kernel(x, table):
    B, L, D = x.shape
    sc = _sc_add(B, L, D)
    out = sc(x.reshape(B * L, D), table)
    return out.reshape(B, L, D)


# speedup vs baseline: 1.0258x; 1.0258x over previous
import functools

import jax
import jax.numpy as jnp
from jax import lax
from jax.experimental import pallas as pl
from jax.experimental.pallas import tpu as pltpu
from jax.experimental.pallas import tpu_sc as plsc

_NC = 2
_NS = 16
_NW = _NC * _NS
_LANES = 16
_NBUF = 4
_AHEAD = 2


def _sc_add(B, L, D):
    rows_per_w = L // _NW
    sub = 8
    n_sub = rows_per_w // sub
    n_steps = B * n_sub
    n_groups = n_steps // _NBUF
    cgroups = D // _LANES
    mesh = plsc.VectorSubcoreMesh(core_axis_name="c", subcore_axis_name="s")

    @functools.partial(
        pl.kernel,
        mesh=mesh,
        out_type=jax.ShapeDtypeStruct((B * L, D), jnp.float32),
        scratch_types=[
            pltpu.VMEM((rows_per_w, D), jnp.float32),
            [pltpu.VMEM((sub, D), jnp.float32) for _ in range(_NBUF)],
            pltpu.SemaphoreType.DMA,
            [pltpu.SemaphoreType.DMA for _ in range(_NBUF)],
            [pltpu.SemaphoreType.DMA for _ in range(_NBUF)],
        ],
    )
    def k(x_hbm, t_hbm, o_hbm, tch, bufs, sem_t, sems_in, sems_out):
        wid = lax.axis_index("s") * _NC + lax.axis_index("c")
        l0 = wid * rows_per_w

        def hbm_row(i):
            b = i // n_sub
            s = i % n_sub
            return b * L + l0 + s * sub

        def issue_in(t, slot):
            return pltpu.async_copy(
                x_hbm.at[pl.ds(hbm_row(t), sub)], bufs[slot], sems_in[slot]
            )

        def wait_in(slot):
            pltpu.make_async_copy(
                x_hbm.at[pl.ds(0, sub)], bufs[slot], sems_in[slot]
            ).wait()

        def wait_out(slot):
            pltpu.make_async_copy(
                bufs[slot], o_hbm.at[pl.ds(0, sub)], sems_out[slot]
            ).wait()

        t_dma = pltpu.async_copy(t_hbm.at[pl.ds(l0, rows_per_w)], tch, sem_t)
        for j in range(_AHEAD):
            issue_in(j, j)
        t_dma.wait()

        def group(g, carry):
            for b in range(_NBUF):
                i = g * _NBUF + b
                slot_next = (b + _AHEAD) % _NBUF

                @pl.when(i < n_steps - _AHEAD)
                def _():
                    @pl.when(i >= _NBUF - _AHEAD)
                    def _():
                        wait_out(slot_next)

                    issue_in(i + _AHEAD, slot_next)

                wait_in(b)
                xb = bufs[b]
                trow0 = (i % n_sub) * sub

                def add_body(j, xb=xb, trow0=trow0):
                    r = j // cgroups
                    c = (j % cgroups) * _LANES
                    plsc.addupdate(
                        xb.at[r, pl.ds(c, _LANES)],
                        tch[trow0 + r, pl.ds(c, _LANES)],
                    )

                plsc.parallel_loop(0, sub * cgroups, unroll=8)(add_body)
                pltpu.async_copy(
                    xb, o_hbm.at[pl.ds(hbm_row(i), sub)], sems_out[b]
                )
            return carry

        lax.fori_loop(0, n_groups, group, 0)
        for b in range(_NBUF):
            wait_out(b)

    return k


def kernel(x, table):
    B, L, D = x.shape
    sc = _sc_add(B, L, D)
    out = sc(x.reshape(B * L, D), table)
    return out.reshape(B, L, D)
